# baseline (speedup 1.0000x reference)
import functools
import math

import jax
import jax.numpy as jnp
from jax import lax
from jax.experimental import pallas as pl
from jax.experimental.pallas import tpu as pltpu
from jax.experimental.pallas import tpu_sc as plsc

_LFREQ = 10000.0
_LANES = 16


def _sinusoidal_pe(length, dm):
    pos = jnp.arange(length, dtype=jnp.float32)[:, None]
    i = jnp.arange(0, dm, 2, dtype=jnp.float32)
    div = jnp.exp(-(jnp.log(_LFREQ)) * i / dm)
    angles = pos * div[None, :]
    pe = jnp.zeros((length, dm), dtype=jnp.float32)
    pe = pe.at[:, 0::2].set(jnp.sin(angles))
    pe = pe.at[:, 1::2].set(jnp.cos(angles))
    return pe


@functools.partial(jax.jit, static_argnames=("bsz", "dm", "length"))
def _embed_sc(idx, pe, table, bsz, dm, length):
    info = plsc.get_sparse_core_info()
    nc, ns = info.num_cores, info.num_subcores
    nw = nc * ns
    n_chunks = bsz // nw
    half = dm // 2
    vecs_per_half = half // _LANES
    scale = jnp.float32(math.sqrt(dm))

    mesh = plsc.VectorSubcoreMesh(core_axis_name="c", subcore_axis_name="s")

    @functools.partial(
        pl.kernel,
        out_type=jax.ShapeDtypeStruct((bsz, length, dm), jnp.float32),
        mesh=mesh,
        scratch_types=[
            pltpu.VMEM((n_chunks, length), jnp.int32),
            pltpu.VMEM((length * dm,), jnp.float32),
            pltpu.VMEM((length, dm), jnp.float32),
            pltpu.VMEM((length, dm), jnp.float32),
            pltpu.VMEM((length, half), jnp.float32),
            pltpu.VMEM((length, half), jnp.float32),
        ]
        + [pltpu.SemaphoreType.DMA] * 4,
    )
    def body(idx_hbm, pe_hbm, table_hbm, out_hbm, idx_v, pe_v,
             x0, x1, y0, y1, g0, g1, o0, o1):
        xs, ys, gs, os_ = (x0, x1), (y0, y1), (g0, g1), (o0, o1)
        wid = lax.axis_index("s") * nc + lax.axis_index("c")
        base_b = wid * n_chunks
        pltpu.sync_copy(pe_hbm, pe_v)
        pltpu.sync_copy(idx_hbm.at[pl.ds(base_b, n_chunks)], idx_v)

        def gather(c, i):
            return pltpu.make_async_copy(
                table_hbm.at[idx_v.at[c]], xs[i], gs[i]
            )

        def out_copy(c, h):
            return pltpu.make_async_copy(
                ys[h],
                out_hbm.at[base_b + c, :, pl.ds(h * half, half)],
                os_[h],
            )

        gather(0, 0).start()

        def step(c, i):
            gather(c, i).wait()

            @pl.when(c + 1 < n_chunks)
            def _():
                gather(c + 1, 1 - i).start()

            x = xs[i]
            for h in range(2):
                @pl.when(c >= 1)
                def _(h=h):
                    out_copy(c - 1, h).wait()

                y = ys[h]

                def row_body(r, _2, h=h, y=y):
                    base_col = h * half
                    pes = [pe_v[pl.ds(r * dm + base_col + j * _LANES, _LANES)]
                           for j in range(vecs_per_half)]
                    for j in range(vecs_per_half):
                        y[r, pl.ds(j * _LANES, _LANES)] = (
                            x[r, pl.ds(base_col + j * _LANES, _LANES)] * scale
                            + pes[j]
                        )
                    return 0

                lax.fori_loop(0, length, row_body, 0)
                out_copy(c, h).start()

        def round_body(k, _):
            step(2 * k, 0)
            step(2 * k + 1, 1)
            return 0

        lax.fori_loop(0, n_chunks // 2, round_body, 0)
        out_copy(n_chunks - 1, 0).wait()
        out_copy(n_chunks - 1, 1).wait()

    return body(idx, pe, table)


def kernel(text, embed_table):
    b, l = text.shape
    v, dm = embed_table.shape
    idx = text.astype(jnp.int32)
    pe = _sinusoidal_pe(l, dm).reshape(-1)
    return _embed_sc(idx, pe, embed_table, b, dm, l)
